# Initial kernel scaffold; baseline (speedup 1.0000x reference)
import jax
import jax.numpy as jnp
from jax.experimental import pallas as pl


def kernel(x, emb_table):
    raise NotImplementedError("write your pallas kernel here")


# speedup vs baseline: 5.2184x; 5.2184x over previous
import functools

import jax
import jax.numpy as jnp
from jax import lax
from jax.experimental import pallas as pl
from jax.experimental.pallas import tpu as pltpu
from jax.experimental.pallas import tpu_sc as plsc

CHUNK = 128


@functools.cache
def _make_gather(B, D):
    info = plsc.get_sparse_core_info()
    NC, NS = info.num_cores, info.num_subcores
    NW = NC * NS
    assert B % (NW * CHUNK) == 0
    b_per_w = B // NW
    n_chunks = b_per_w // CHUNK

    mesh = plsc.VectorSubcoreMesh(core_axis_name="c", subcore_axis_name="s")

    @functools.partial(
        pl.kernel,
        mesh=mesh,
        out_type=jax.ShapeDtypeStruct((B, D), jnp.float32),
        scratch_types=[
            pltpu.VMEM((n_chunks, CHUNK), jnp.int32),
            pltpu.VMEM((CHUNK, D), jnp.float32),
            pltpu.SemaphoreType.DMA,
        ],
        compiler_params=pltpu.CompilerParams(use_tc_tiling_on_sc=False),
    )
    def gather_kernel(table_hbm, idx_hbm, out_hbm, idx_v, rows_v, sem):
        wid = lax.axis_index("s") * NC + lax.axis_index("c")
        base = wid * b_per_w
        pltpu.sync_copy(idx_hbm.at[pl.ds(wid * n_chunks, n_chunks)], idx_v)

        def chunk_body(j, carry):
            pltpu.async_copy(table_hbm.at[idx_v.at[j]], rows_v, sem).wait()
            pltpu.sync_copy(rows_v, out_hbm.at[pl.ds(base + j * CHUNK, CHUNK)])
            return carry

        lax.fori_loop(0, n_chunks, chunk_body, 0)

    return gather_kernel


def kernel(x, emb_table):
    B = x.size
    D = emb_table.shape[1]
    idx = x.reshape(B // CHUNK, CHUNK).astype(jnp.int32)
    out = _make_gather(B, D)(emb_table, idx)
    return out.reshape(x.shape + (D,))


# speedup vs baseline: 6.2344x; 1.1947x over previous
import functools

import jax
import jax.numpy as jnp
from jax import lax
from jax.experimental import pallas as pl
from jax.experimental.pallas import tpu as pltpu
from jax.experimental.pallas import tpu_sc as plsc

CHUNK = 128
S = 2
SUPER = S * CHUNK
NBUF = 4


@functools.cache
def _make_gather(B, D):
    info = plsc.get_sparse_core_info()
    NC, NS = info.num_cores, info.num_subcores
    NW = NC * NS
    assert B % (NW * SUPER) == 0
    b_per_w = B // NW
    n_chunks = b_per_w // CHUNK
    n_super = b_per_w // SUPER
    assert n_super % NBUF == 0

    mesh = plsc.VectorSubcoreMesh(core_axis_name="c", subcore_axis_name="s")

    @functools.partial(
        pl.kernel,
        mesh=mesh,
        out_type=jax.ShapeDtypeStruct((B, D), jnp.float32),
        scratch_types=[
            pltpu.VMEM((n_chunks, CHUNK), jnp.int32),
            pltpu.VMEM((NBUF, SUPER, D), jnp.float32),
        ]
        + [pltpu.SemaphoreType.DMA] * (2 * NBUF),
        compiler_params=pltpu.CompilerParams(use_tc_tiling_on_sc=False),
    )
    def gather_kernel(table_hbm, idx_hbm, out_hbm, idx_v, rows_v, *sems):
        gsem = sems[:NBUF]
        ssem = sems[NBUF:]
        wid = lax.axis_index("s") * NC + lax.axis_index("c")
        base = wid * b_per_w
        pltpu.sync_copy(idx_hbm.at[pl.ds(wid * n_chunks, n_chunks)], idx_v)

        def fire_gathers(g, b):
            for s in range(S):
                pltpu.async_copy(
                    table_hbm.at[idx_v.at[g * S + s]],
                    rows_v.at[b, pl.ds(s * CHUNK, CHUNK)],
                    gsem[b],
                )

        def wait_gathers(g, b):
            for s in range(S):
                pltpu.make_async_copy(
                    table_hbm.at[idx_v.at[g * S + s]],
                    rows_v.at[b, pl.ds(s * CHUNK, CHUNK)],
                    gsem[b],
                ).wait()

        def fire_store(g, b):
            pltpu.async_copy(
                rows_v.at[b], out_hbm.at[pl.ds(base + g * SUPER, SUPER)], ssem[b]
            )

        def wait_store(g, b):
            pltpu.make_async_copy(
                rows_v.at[b], out_hbm.at[pl.ds(base + g * SUPER, SUPER)], ssem[b]
            ).wait()

        for b in range(NBUF):
            fire_gathers(b, b)

        def outer_body(outer, carry):
            for db in range(NBUF):
                g = outer * NBUF + db
                bp = (db - 1) % NBUF
                gp = g - 1 + NBUF

                @pl.when((g >= 1) & (gp < n_super))
                def _():
                    wait_store(gp - NBUF, bp)
                    fire_gathers(gp, bp)

                wait_gathers(g, db)
                fire_store(g, db)
            return carry

        lax.fori_loop(0, n_super // NBUF, outer_body, 0)

        for b in range(NBUF):
            wait_store(n_super - NBUF + b, b)

    return gather_kernel


def kernel(x, emb_table):
    B = x.size
    D = emb_table.shape[1]
    idx = x.reshape(B // CHUNK, CHUNK).astype(jnp.int32)
    out = _make_gather(B, D)(emb_table, idx)
    return out.reshape(x.shape + (D,))


# speedup vs baseline: 6.7460x; 1.0820x over previous
import functools

import jax
import jax.numpy as jnp
from jax import lax
from jax.experimental import pallas as pl
from jax.experimental.pallas import tpu as pltpu
from jax.experimental.pallas import tpu_sc as plsc

CHUNK = 128
S = 2
SUPER = S * CHUNK
NBUF = 4


@functools.cache
def _make_gather(H, Bb, D):
    B = H * Bb
    info = plsc.get_sparse_core_info()
    NC, NS = info.num_cores, info.num_subcores
    NW = NC * NS
    assert B % (NW * SUPER) == 0 and Bb % SUPER == 0
    b_per_w = B // NW
    n_chunks = b_per_w // CHUNK
    n_super = b_per_w // SUPER
    assert n_super % NBUF == 0

    mesh = plsc.VectorSubcoreMesh(core_axis_name="c", subcore_axis_name="s")

    @functools.partial(
        pl.kernel,
        mesh=mesh,
        out_type=jax.ShapeDtypeStruct((H, Bb, D), jnp.float32),
        scratch_types=[
            pltpu.VMEM((n_chunks, CHUNK), jnp.int32),
            pltpu.VMEM((NBUF, SUPER, D), jnp.float32),
        ]
        + [pltpu.SemaphoreType.DMA] * (2 * NBUF),
        compiler_params=pltpu.CompilerParams(use_tc_tiling_on_sc=False),
    )
    def gather_kernel(table_hbm, idx_hbm, out_hbm, idx_v, rows_v, *sems):
        gsem = sems[:NBUF]
        ssem = sems[NBUF:]
        wid = lax.axis_index("s") * NC + lax.axis_index("c")
        base = wid * b_per_w
        pltpu.sync_copy(idx_hbm.at[pl.ds(wid * n_chunks, n_chunks)], idx_v)

        def out_view(g, b):
            row = base + g * SUPER
            return out_hbm.at[row // Bb, pl.ds(row % Bb, SUPER)]

        def fire_gathers(g, b):
            for s in range(S):
                pltpu.async_copy(
                    table_hbm.at[idx_v.at[g * S + s]],
                    rows_v.at[b, pl.ds(s * CHUNK, CHUNK)],
                    gsem[b],
                )

        def wait_gathers(g, b):
            for s in range(S):
                pltpu.make_async_copy(
                    table_hbm.at[idx_v.at[g * S + s]],
                    rows_v.at[b, pl.ds(s * CHUNK, CHUNK)],
                    gsem[b],
                ).wait()

        def fire_store(g, b):
            pltpu.async_copy(rows_v.at[b], out_view(g, b), ssem[b])

        def wait_store(g, b):
            pltpu.make_async_copy(rows_v.at[b], out_view(g, b), ssem[b]).wait()

        for b in range(NBUF):
            fire_gathers(b, b)

        def outer_body(outer, carry):
            for db in range(NBUF):
                g = outer * NBUF + db
                bp = (db - 1) % NBUF
                gp = g - 1 + NBUF

                @pl.when((g >= 1) & (gp < n_super))
                def _():
                    wait_store(gp - NBUF, bp)
                    fire_gathers(gp, bp)

                wait_gathers(g, db)
                fire_store(g, db)
            return carry

        lax.fori_loop(0, n_super // NBUF, outer_body, 0)

        for b in range(NBUF):
            wait_store(n_super - NBUF + b, b)

    return gather_kernel


def kernel(x, emb_table):
    Bb, H = x.shape
    D = emb_table.shape[1]
    idx = jnp.swapaxes(x, 0, 1).reshape(x.size // CHUNK, CHUNK).astype(jnp.int32)
    out = _make_gather(H, Bb, D)(emb_table, idx)
    return jnp.swapaxes(out, 0, 1)
